# baseline (speedup 1.0000x reference)
import functools

import numpy as np
import jax
import jax.numpy as jnp
from jax import lax
from jax.experimental import pallas as pl
from jax.experimental.pallas import tpu as pltpu


def _banded_weight(w_taps, win, wout):
    cin, cout = w_taps.shape[1], w_taps.shape[2]
    per_kh = []
    for kh in range(3):
        acc = jnp.zeros((win, cin, wout, cout), jnp.float32)
        for kw in range(3):
            sel = jnp.eye(win, wout, k=-kw, dtype=jnp.float32)
            tap = w_taps[kh * 3 + kw].astype(jnp.float32)
            acc = acc + jnp.einsum("io,cd->icod", sel, tap)
        per_kh.append(acc.reshape(win * cin, wout * cout))
    return jnp.concatenate(per_kh, axis=0).astype(jnp.bfloat16)


def _banded_weight_c1(wk1, win, wout):
    return _banded_weight(wk1[:, None, :], win, wout)


def _pool_max_lanes(y, groups, pool, c):
    outs = []
    for g in range(groups):
        m = y[:, (g * pool) * c:(g * pool) * c + c]
        for p in range(1, pool):
            m = jnp.maximum(m, y[:, (g * pool + p) * c:(g * pool + p) * c + c])
        outs.append(m)
    return jnp.concatenate(outs, axis=1)


def _conv_banded_grouped(cp, w, t, wout, *, T):
    outs = []
    for g in range(wout // 2):
        a = jnp.concatenate(
            [cp[kh:kh + T, g * 256:g * 256 + 512] for kh in range(3)], axis=1)
        y = jnp.dot(a, w, preferred_element_type=jnp.float32)
        outs.append(y)
    y = jnp.concatenate(outs, axis=1)
    y = _pool_max_lanes(y, wout // 2, 2, 128)
    return jnp.maximum(y + t, 0.0).astype(jnp.bfloat16)


def _conv_stack_kernel(xp_ref, w1_ref, t1_ref, w2_ref, t2_ref,
                       w3_ref, t3_ref, gw_ref, gb_ref, o_ref, *, T, bt):
    w1 = w1_ref[...]
    w2 = w2_ref[...]
    w3 = w3_ref[...]
    gw = gw_ref[...]

    for i in range(bt):
        xi = xp_ref[i]
        a = jnp.concatenate([xi[0:T], xi[1:T + 1], xi[2:T + 2]],
                            axis=1).astype(jnp.bfloat16)
        y = jnp.dot(a, w1, preferred_element_type=jnp.float32)
        y = _pool_max_lanes(y, 8, 5, 128)
        c1 = jnp.maximum(y + t1_ref[...], 0.0).astype(jnp.bfloat16)
        c1 = jnp.pad(c1, ((1, 1), (128, 128)))

        c2 = _conv_banded_grouped(c1, w2, t2_ref[...], 8, T=T)
        c2 = jnp.pad(c2, ((1, 1), (128, 128)))

        c3 = _conv_banded_grouped(c2, w3, t3_ref[...], 4, T=T)
        xg = jnp.dot(c3, gw, preferred_element_type=jnp.float32) + gb_ref[...]
        o_ref[:, i, :] = xg.astype(o_ref.dtype)


def _conv_stack(x, wk1, s1, t1, wk2, s2, t2, wk3, s3, t3, g1w, g1b, *, bt=16):
    B, T, F = x.shape
    xp = jnp.pad(x, ((0, 0), (1, 1), (1, 1)))
    GH = g1w.shape[1]

    w1 = _banded_weight_c1(wk1.astype(jnp.float32) * s1, F + 2, F)
    w2 = _banded_weight(wk2.astype(jnp.float32) * s2, 4, 2)
    w3 = _banded_weight(wk3.astype(jnp.float32) * s3, 4, 2)
    t1t = jnp.tile(t1, (1, F // 5))
    t2t = jnp.tile(t2, (1, 4))
    t3t = jnp.tile(t3, (1, 2))

    kern = functools.partial(_conv_stack_kernel, T=T, bt=bt)
    full = lambda shape: pl.BlockSpec(shape, lambda b: tuple(0 for _ in shape))
    return pl.pallas_call(
        kern,
        out_shape=jax.ShapeDtypeStruct((T, B, GH), jnp.bfloat16),
        grid=(B // bt,),
        in_specs=[pl.BlockSpec((bt, T + 2, F + 2), lambda b: (b, 0, 0)),
                  full((126, 5120)), full((1, 1024)),
                  full((1536, 256)), full((1, 512)),
                  full((1536, 256)), full((1, 256)),
                  full((256, GH)), full((1, GH))],
        out_specs=pl.BlockSpec((T, bt, GH), lambda b: (0, b, 0)),
        compiler_params=pltpu.CompilerParams(
            dimension_semantics=("parallel",),
            vmem_limit_bytes=60 * 1024 * 1024),
    )(xp, w1, t1t, w2, t2t, w3, t3t, g1w.astype(jnp.bfloat16), g1b)


def _gru_steps(xg_ref, w_bd, b_hh, h_out_ref, *, T, H, Bt, unroll):
    H2, H4 = 2 * H, 4 * H
    lane = lax.broadcasted_iota(jnp.int32, (1, 6 * H), 1)
    fwd_lane = (lane % H2) < H

    def step(t, h):
        tb = T - 1 - t
        xgm = jnp.where(fwd_lane, xg_ref[t], xg_ref[tb])
        hg = jnp.dot(h.astype(w_bd.dtype), w_bd,
                     preferred_element_type=jnp.float32) + b_hh
        rz = jax.nn.sigmoid(xgm[:, 0:H4] + hg[:, 0:H4])
        n = jnp.tanh(xgm[:, H4:] + rz[:, 0:H2] * hg[:, H4:])
        z = rz[:, H2:H4]
        h_new = (1.0 - z) * n + z * h
        h_out_ref[pl.ds(t, 1), :, 0:H] = h_new[None, :, 0:H]
        h_out_ref[pl.ds(tb, 1), :, H:H2] = h_new[None, :, H:H2]
        return h_new

    lax.fori_loop(0, T, step, jnp.zeros((Bt, H2), jnp.float32), unroll=unroll)


def _head_kernel(h_ref, w1_ref, b1_ref, w2_ref, b2_ref, o_ref, *, n_fc):
    h1 = jnp.dot(h_ref[...], w1_ref[...],
                 preferred_element_type=jnp.float32) + b1_ref[...]
    h1 = jnp.maximum(h1, 0.0)
    y = jnp.dot(h1.astype(jnp.bfloat16), w2_ref[...],
                preferred_element_type=jnp.float32) + b2_ref[...]
    o_ref[...] = jax.nn.sigmoid(y)


def _recurrent_kernel(xg1_hbm, g1wbd_ref, g1bhh_ref,
                      g2w_ref, g2b_ref, g2wbd_ref, g2bhh_ref, o_ref,
                      xg1_ref, h_ref, sem, *, T, H, unroll, chunks=8):
    Bt = xg1_ref.shape[1]
    Tc = T // chunks

    b = pl.program_id(0)
    cp = pltpu.make_async_copy(
        xg1_hbm.at[:, pl.ds(b * Bt, Bt), :], xg1_ref, sem)
    cp.start()
    cp.wait()

    _gru_steps(xg1_ref, g1wbd_ref[...], g1bhh_ref[...], h_ref,
               T=T, H=H, Bt=Bt, unroll=unroll)

    for c in range(chunks):
        hc = h_ref[c * Tc:(c + 1) * Tc].reshape(Tc * Bt, 2 * H)
        xg2 = jnp.dot(hc.astype(jnp.bfloat16), g2w_ref[...],
                      preferred_element_type=jnp.float32) + g2b_ref[...]
        xg1_ref[c * Tc:(c + 1) * Tc] = (
            xg2.reshape(Tc, Bt, 6 * H).astype(jnp.bfloat16))
    _gru_steps(xg1_ref, g2wbd_ref[...], g2bhh_ref[...], h_ref,
               T=T, H=H, Bt=Bt, unroll=unroll)

    for c in range(chunks):
        o_ref[c * Tc:(c + 1) * Tc] = (
            h_ref[c * Tc:(c + 1) * Tc].astype(jnp.bfloat16))


def _recurrent_stack(xg1, g1_wbd, g1_bhh, g2_wiht, g2_bih, g2_wbd, g2_bhh,
                     fc1w, fc1b, fc2w, fc2b, *, bt=128):
    T, B, _ = xg1.shape
    H = g1_wbd.shape[0] // 2
    n_fc = fc1w.shape[1]
    n_out = fc2w.shape[1]

    full = lambda shape: pl.BlockSpec(shape, lambda b: tuple(0 for _ in shape))
    h2 = pl.pallas_call(
        functools.partial(_recurrent_kernel, T=T, H=H, unroll=8),
        out_shape=jax.ShapeDtypeStruct((T, B, 2 * H), jnp.bfloat16),
        grid=(B // bt,),
        in_specs=[pl.BlockSpec(memory_space=pl.ANY),
                  full((2 * H, 6 * H)), full((1, 6 * H)),
                  full((2 * H, 6 * H)), full((1, 6 * H)),
                  full((2 * H, 6 * H)), full((1, 6 * H))],
        out_specs=pl.BlockSpec((T, bt, 2 * H), lambda b: (0, b, 0)),
        scratch_shapes=[pltpu.VMEM((T, bt, 6 * H), jnp.bfloat16),
                        pltpu.VMEM((T, bt, 2 * H), jnp.float32),
                        pltpu.SemaphoreType.DMA],
        compiler_params=pltpu.CompilerParams(
            dimension_semantics=("parallel",),
            vmem_limit_bytes=62 * 1024 * 1024),
    )(xg1, g1_wbd, g1_bhh, g2_wiht, g2_bih, g2_wbd, g2_bhh)

    m = T * B
    tile_m = 4096
    out2 = pl.pallas_call(
        functools.partial(_head_kernel, n_fc=n_fc),
        out_shape=jax.ShapeDtypeStruct((m, n_out), jnp.float32),
        grid=(m // tile_m,),
        in_specs=[pl.BlockSpec((tile_m, 2 * H), lambda i: (i, 0)),
                  full((2 * H, n_fc)), full((1, n_fc)),
                  full((n_fc, n_out)), full((1, n_out))],
        out_specs=pl.BlockSpec((tile_m, n_out), lambda i: (i, 0)),
        compiler_params=pltpu.CompilerParams(
            dimension_semantics=("parallel",)),
    )(h2.reshape(m, 2 * H), fc1w, fc1b, fc2w, fc2b)
    return out2.reshape(T, B, n_out)


def kernel(x, wk1, s1, t1, wk2, s2, t2, wk3, s3, t3,
           g1_wiht, g1_bih, g1_wbd, g1_bhh,
           g2_wiht, g2_bih, g2_wbd, g2_bhh,
           fc1w, fc1b, fc2w, fc2b):
    H = g1_wbd.shape[0] // 2
    p = np.concatenate([np.arange(0, H), np.arange(3 * H, 4 * H),
                        np.arange(H, 2 * H), np.arange(4 * H, 5 * H),
                        np.arange(2 * H, 3 * H), np.arange(5 * H, 6 * H)])
    xg1 = _conv_stack(x, wk1, s1, t1, wk2, s2, t2, wk3, s3, t3,
                      g1_wiht[:, p], g1_bih[:, p])
    out_tm = _recurrent_stack(xg1, g1_wbd[:, p], g1_bhh[:, p],
                              g2_wiht[:, p], g2_bih[:, p],
                              g2_wbd[:, p], g2_bhh[:, p],
                              fc1w, fc1b, fc2w, fc2b)
    return jnp.transpose(out_tm, (1, 0, 2))


# speedup vs baseline: 1.0090x; 1.0090x over previous
import functools

import numpy as np
import jax
import jax.numpy as jnp
from jax import lax
from jax.experimental import pallas as pl
from jax.experimental.pallas import tpu as pltpu


def _banded_weight(w_taps, win, wout):
    cin, cout = w_taps.shape[1], w_taps.shape[2]
    per_kh = []
    for kh in range(3):
        acc = jnp.zeros((win, cin, wout, cout), jnp.float32)
        for kw in range(3):
            sel = jnp.eye(win, wout, k=-kw, dtype=jnp.float32)
            tap = w_taps[kh * 3 + kw].astype(jnp.float32)
            acc = acc + jnp.einsum("io,cd->icod", sel, tap)
        per_kh.append(acc.reshape(win * cin, wout * cout))
    return jnp.concatenate(per_kh, axis=0).astype(jnp.bfloat16)


def _banded_weight_c1(wk1, win, wout):
    return _banded_weight(wk1[:, None, :], win, wout)


def _pool_max_lanes(y, groups, pool, c):
    outs = []
    for g in range(groups):
        m = y[:, (g * pool) * c:(g * pool) * c + c]
        for p in range(1, pool):
            m = jnp.maximum(m, y[:, (g * pool + p) * c:(g * pool + p) * c + c])
        outs.append(m)
    return jnp.concatenate(outs, axis=1)


def _conv_banded_grouped(cp, w, t, wout, *, T):
    outs = []
    for g in range(wout // 2):
        a = jnp.concatenate(
            [cp[kh:kh + T, g * 256:g * 256 + 512] for kh in range(3)], axis=1)
        y = jnp.dot(a, w, preferred_element_type=jnp.float32)
        outs.append(y)
    y = jnp.concatenate(outs, axis=1)
    y = _pool_max_lanes(y, wout // 2, 2, 128)
    return jnp.maximum(y + t, 0.0).astype(jnp.bfloat16)


def _conv_stack_kernel(xp_ref, w1_ref, t1_ref, w2_ref, t2_ref,
                       w3_ref, t3_ref, gw_ref, gb_ref, o_ref, *, T, bt):
    w1 = w1_ref[...]
    w2 = w2_ref[...]
    w3 = w3_ref[...]
    gw = gw_ref[...]

    for i in range(bt):
        xi = xp_ref[i]
        a = jnp.concatenate([xi[0:T], xi[1:T + 1], xi[2:T + 2]],
                            axis=1).astype(jnp.bfloat16)
        y = jnp.dot(a, w1, preferred_element_type=jnp.float32)
        y = _pool_max_lanes(y, 8, 5, 128)
        c1 = jnp.maximum(y + t1_ref[...], 0.0).astype(jnp.bfloat16)
        c1 = jnp.pad(c1, ((1, 1), (128, 128)))

        c2 = _conv_banded_grouped(c1, w2, t2_ref[...], 8, T=T)
        c2 = jnp.pad(c2, ((1, 1), (128, 128)))

        c3 = _conv_banded_grouped(c2, w3, t3_ref[...], 4, T=T)
        xg = jnp.dot(c3, gw, preferred_element_type=jnp.float32) + gb_ref[...]
        o_ref[:, i, :] = xg.astype(o_ref.dtype)


def _conv_stack(x, wk1, s1, t1, wk2, s2, t2, wk3, s3, t3, g1w, g1b, *, bt=8):
    B, T, F = x.shape
    xp = jnp.pad(x, ((0, 0), (1, 1), (1, 1)))
    GH = g1w.shape[1]

    w1 = _banded_weight_c1(wk1.astype(jnp.float32) * s1, F + 2, F)
    w2 = _banded_weight(wk2.astype(jnp.float32) * s2, 4, 2)
    w3 = _banded_weight(wk3.astype(jnp.float32) * s3, 4, 2)
    t1t = jnp.tile(t1, (1, F // 5))
    t2t = jnp.tile(t2, (1, 4))
    t3t = jnp.tile(t3, (1, 2))

    kern = functools.partial(_conv_stack_kernel, T=T, bt=bt)
    full = lambda shape: pl.BlockSpec(shape, lambda b: tuple(0 for _ in shape))
    return pl.pallas_call(
        kern,
        out_shape=jax.ShapeDtypeStruct((T, B, GH), jnp.bfloat16),
        grid=(B // bt,),
        in_specs=[pl.BlockSpec((bt, T + 2, F + 2), lambda b: (b, 0, 0)),
                  full((126, 5120)), full((1, 1024)),
                  full((1536, 256)), full((1, 512)),
                  full((1536, 256)), full((1, 256)),
                  full((256, GH)), full((1, GH))],
        out_specs=pl.BlockSpec((T, bt, GH), lambda b: (0, b, 0)),
        compiler_params=pltpu.CompilerParams(
            dimension_semantics=("parallel",),
            vmem_limit_bytes=60 * 1024 * 1024),
    )(xp, w1, t1t, w2, t2t, w3, t3t, g1w.astype(jnp.bfloat16), g1b)


def _gru_steps(xg_ref, w_bd, b_hh, h_out_ref, *, T, H, Bt, unroll):
    H2, H4 = 2 * H, 4 * H
    lane = lax.broadcasted_iota(jnp.int32, (1, 6 * H), 1)
    fwd_lane = (lane % H2) < H

    def step(t, h):
        tb = T - 1 - t
        xgm = jnp.where(fwd_lane, xg_ref[t], xg_ref[tb])
        hg = jnp.dot(h.astype(w_bd.dtype), w_bd,
                     preferred_element_type=jnp.float32) + b_hh
        rz = jax.nn.sigmoid(xgm[:, 0:H4] + hg[:, 0:H4])
        n = jnp.tanh(xgm[:, H4:] + rz[:, 0:H2] * hg[:, H4:])
        z = rz[:, H2:H4]
        h_new = (1.0 - z) * n + z * h
        h_out_ref[pl.ds(t, 1), :, 0:H] = h_new[None, :, 0:H]
        h_out_ref[pl.ds(tb, 1), :, H:H2] = h_new[None, :, H:H2]
        return h_new

    lax.fori_loop(0, T, step, jnp.zeros((Bt, H2), jnp.float32), unroll=unroll)


def _head_kernel(h_ref, w1_ref, b1_ref, w2_ref, b2_ref, o_ref, *, n_fc):
    h1 = jnp.dot(h_ref[...], w1_ref[...],
                 preferred_element_type=jnp.float32) + b1_ref[...]
    h1 = jnp.maximum(h1, 0.0)
    y = jnp.dot(h1.astype(jnp.bfloat16), w2_ref[...],
                preferred_element_type=jnp.float32) + b2_ref[...]
    o_ref[...] = jax.nn.sigmoid(y)


def _recurrent_kernel(xg1_hbm, g1wbd_ref, g1bhh_ref,
                      g2w_ref, g2b_ref, g2wbd_ref, g2bhh_ref, o_ref,
                      xg1_ref, h_ref, sem, *, T, H, unroll, chunks=8):
    Bt = xg1_ref.shape[1]
    Tc = T // chunks

    b = pl.program_id(0)
    cp = pltpu.make_async_copy(
        xg1_hbm.at[:, pl.ds(b * Bt, Bt), :], xg1_ref, sem)
    cp.start()
    cp.wait()

    _gru_steps(xg1_ref, g1wbd_ref[...], g1bhh_ref[...], h_ref,
               T=T, H=H, Bt=Bt, unroll=unroll)

    for c in range(chunks):
        hc = h_ref[c * Tc:(c + 1) * Tc].reshape(Tc * Bt, 2 * H)
        xg2 = jnp.dot(hc.astype(jnp.bfloat16), g2w_ref[...],
                      preferred_element_type=jnp.float32) + g2b_ref[...]
        xg1_ref[c * Tc:(c + 1) * Tc] = (
            xg2.reshape(Tc, Bt, 6 * H).astype(jnp.bfloat16))
    _gru_steps(xg1_ref, g2wbd_ref[...], g2bhh_ref[...], h_ref,
               T=T, H=H, Bt=Bt, unroll=unroll)

    for c in range(chunks):
        o_ref[c * Tc:(c + 1) * Tc] = (
            h_ref[c * Tc:(c + 1) * Tc].astype(jnp.bfloat16))


def _recurrent_stack(xg1, g1_wbd, g1_bhh, g2_wiht, g2_bih, g2_wbd, g2_bhh,
                     fc1w, fc1b, fc2w, fc2b, *, bt=128):
    T, B, _ = xg1.shape
    H = g1_wbd.shape[0] // 2
    n_fc = fc1w.shape[1]
    n_out = fc2w.shape[1]

    full = lambda shape: pl.BlockSpec(shape, lambda b: tuple(0 for _ in shape))
    h2 = pl.pallas_call(
        functools.partial(_recurrent_kernel, T=T, H=H, unroll=8),
        out_shape=jax.ShapeDtypeStruct((T, B, 2 * H), jnp.bfloat16),
        grid=(B // bt,),
        in_specs=[pl.BlockSpec(memory_space=pl.ANY),
                  full((2 * H, 6 * H)), full((1, 6 * H)),
                  full((2 * H, 6 * H)), full((1, 6 * H)),
                  full((2 * H, 6 * H)), full((1, 6 * H))],
        out_specs=pl.BlockSpec((T, bt, 2 * H), lambda b: (0, b, 0)),
        scratch_shapes=[pltpu.VMEM((T, bt, 6 * H), jnp.bfloat16),
                        pltpu.VMEM((T, bt, 2 * H), jnp.float32),
                        pltpu.SemaphoreType.DMA],
        compiler_params=pltpu.CompilerParams(
            dimension_semantics=("parallel",),
            vmem_limit_bytes=62 * 1024 * 1024),
    )(xg1, g1_wbd, g1_bhh, g2_wiht, g2_bih, g2_wbd, g2_bhh)

    m = T * B
    tile_m = 4096
    out2 = pl.pallas_call(
        functools.partial(_head_kernel, n_fc=n_fc),
        out_shape=jax.ShapeDtypeStruct((m, n_out), jnp.float32),
        grid=(m // tile_m,),
        in_specs=[pl.BlockSpec((tile_m, 2 * H), lambda i: (i, 0)),
                  full((2 * H, n_fc)), full((1, n_fc)),
                  full((n_fc, n_out)), full((1, n_out))],
        out_specs=pl.BlockSpec((tile_m, n_out), lambda i: (i, 0)),
        compiler_params=pltpu.CompilerParams(
            dimension_semantics=("parallel",)),
    )(h2.reshape(m, 2 * H), fc1w, fc1b, fc2w, fc2b)
    return out2.reshape(T, B, n_out)


def kernel(x, wk1, s1, t1, wk2, s2, t2, wk3, s3, t3,
           g1_wiht, g1_bih, g1_wbd, g1_bhh,
           g2_wiht, g2_bih, g2_wbd, g2_bhh,
           fc1w, fc1b, fc2w, fc2b):
    H = g1_wbd.shape[0] // 2
    p = np.concatenate([np.arange(0, H), np.arange(3 * H, 4 * H),
                        np.arange(H, 2 * H), np.arange(4 * H, 5 * H),
                        np.arange(2 * H, 3 * H), np.arange(5 * H, 6 * H)])
    xg1 = _conv_stack(x, wk1, s1, t1, wk2, s2, t2, wk3, s3, t3,
                      g1_wiht[:, p], g1_bih[:, p])
    out_tm = _recurrent_stack(xg1, g1_wbd[:, p], g1_bhh[:, p],
                              g2_wiht[:, p], g2_bih[:, p],
                              g2_wbd[:, p], g2_bhh[:, p],
                              fc1w, fc1b, fc2w, fc2b)
    return jnp.transpose(out_tm, (1, 0, 2))


# speedup vs baseline: 1.0164x; 1.0073x over previous
import functools

import numpy as np
import jax
import jax.numpy as jnp
from jax import lax
from jax.experimental import pallas as pl
from jax.experimental.pallas import tpu as pltpu


def _banded_weight(w_taps, win, wout):
    cin, cout = w_taps.shape[1], w_taps.shape[2]
    per_kh = []
    for kh in range(3):
        acc = jnp.zeros((win, cin, wout, cout), jnp.float32)
        for kw in range(3):
            sel = jnp.eye(win, wout, k=-kw, dtype=jnp.float32)
            tap = w_taps[kh * 3 + kw].astype(jnp.float32)
            acc = acc + jnp.einsum("io,cd->icod", sel, tap)
        per_kh.append(acc.reshape(win * cin, wout * cout))
    return jnp.concatenate(per_kh, axis=0).astype(jnp.bfloat16)


def _banded_weight_c1(wk1, win, wout):
    return _banded_weight(wk1[:, None, :], win, wout)


def _pool_max_lanes(y, groups, pool, c):
    outs = []
    for g in range(groups):
        m = y[:, (g * pool) * c:(g * pool) * c + c]
        for p in range(1, pool):
            m = jnp.maximum(m, y[:, (g * pool + p) * c:(g * pool + p) * c + c])
        outs.append(m)
    return jnp.concatenate(outs, axis=1)


def _conv_banded_grouped(cp, w, t, wout, *, T):
    outs = []
    for g in range(wout // 2):
        a = jnp.concatenate(
            [cp[kh:kh + T, g * 256:g * 256 + 512] for kh in range(3)], axis=1)
        y = jnp.dot(a, w, preferred_element_type=jnp.float32)
        outs.append(y)
    y = jnp.concatenate(outs, axis=1)
    y = _pool_max_lanes(y, wout // 2, 2, 128)
    return jnp.maximum(y + t, 0.0).astype(jnp.bfloat16)


def _conv_stack_kernel(xp_ref, w1_ref, t1_ref, w2_ref, t2_ref,
                       w3_ref, t3_ref, gw_ref, gb_ref, o_ref, *, T, bt):
    w1 = w1_ref[...]
    w2 = w2_ref[...]
    w3 = w3_ref[...]
    gw = gw_ref[...]

    for i in range(bt):
        xi = xp_ref[i]
        a = jnp.concatenate([xi[0:T], xi[1:T + 1], xi[2:T + 2]],
                            axis=1).astype(jnp.bfloat16)
        y = jnp.dot(a, w1, preferred_element_type=jnp.float32)
        y = _pool_max_lanes(y, 8, 5, 128)
        c1 = jnp.maximum(y + t1_ref[...], 0.0).astype(jnp.bfloat16)
        c1 = jnp.pad(c1, ((1, 1), (128, 128)))

        c2 = _conv_banded_grouped(c1, w2, t2_ref[...], 8, T=T)
        c2 = jnp.pad(c2, ((1, 1), (128, 128)))

        c3 = _conv_banded_grouped(c2, w3, t3_ref[...], 4, T=T)
        xg = jnp.dot(c3, gw, preferred_element_type=jnp.float32) + gb_ref[...]
        o_ref[:, i, :] = xg.astype(o_ref.dtype)


def _conv_stack(x, wk1, s1, t1, wk2, s2, t2, wk3, s3, t3, g1w, g1b, *, bt=8):
    B, T, F = x.shape
    xp = jnp.pad(x, ((0, 0), (1, 1), (1, 1)))
    GH = g1w.shape[1]

    w1 = _banded_weight_c1(wk1.astype(jnp.float32) * s1, F + 2, F)
    w2 = _banded_weight(wk2.astype(jnp.float32) * s2, 4, 2)
    w3 = _banded_weight(wk3.astype(jnp.float32) * s3, 4, 2)
    t1t = jnp.tile(t1, (1, F // 5))
    t2t = jnp.tile(t2, (1, 4))
    t3t = jnp.tile(t3, (1, 2))

    kern = functools.partial(_conv_stack_kernel, T=T, bt=bt)
    full = lambda shape: pl.BlockSpec(shape, lambda b: tuple(0 for _ in shape))
    return pl.pallas_call(
        kern,
        out_shape=jax.ShapeDtypeStruct((T, B, GH), jnp.bfloat16),
        grid=(B // bt,),
        in_specs=[pl.BlockSpec((bt, T + 2, F + 2), lambda b: (b, 0, 0)),
                  full((126, 5120)), full((1, 1024)),
                  full((1536, 256)), full((1, 512)),
                  full((1536, 256)), full((1, 256)),
                  full((256, GH)), full((1, GH))],
        out_specs=pl.BlockSpec((T, bt, GH), lambda b: (0, b, 0)),
        compiler_params=pltpu.CompilerParams(
            dimension_semantics=("parallel",),
            vmem_limit_bytes=60 * 1024 * 1024),
    )(xp, w1, t1t, w2, t2t, w3, t3t, g1w.astype(jnp.bfloat16), g1b)


def _gru_steps(xg_ref, w_bd, b_hh, h_out_ref, *, T, H, Bt, unroll):
    H2, H4 = 2 * H, 4 * H
    lane = lax.broadcasted_iota(jnp.int32, (1, 6 * H), 1)
    fwd_lane = (lane % H2) < H

    def step(t, h):
        tb = T - 1 - t
        xgm = jnp.where(fwd_lane, xg_ref[t], xg_ref[tb])
        hg = jnp.dot(h, w_bd, preferred_element_type=jnp.float32) + b_hh
        rz = jax.nn.sigmoid(xgm[:, 0:H4] + hg[:, 0:H4])
        n = jnp.tanh(xgm[:, H4:] + rz[:, 0:H2] * hg[:, H4:])
        z = rz[:, H2:H4]
        h_new = (1.0 - z) * n + z * h
        h_out_ref[pl.ds(t, 1), :, 0:H] = h_new[None, :, 0:H]
        h_out_ref[pl.ds(tb, 1), :, H:H2] = h_new[None, :, H:H2]
        return h_new

    lax.fori_loop(0, T, step, jnp.zeros((Bt, H2), jnp.float32), unroll=unroll)


def _head_kernel(h_ref, w1_ref, b1_ref, w2_ref, b2_ref, o_ref, *, n_fc):
    h1 = jnp.dot(h_ref[...], w1_ref[...],
                 preferred_element_type=jnp.float32) + b1_ref[...]
    h1 = jnp.maximum(h1, 0.0)
    y = jnp.dot(h1.astype(jnp.bfloat16), w2_ref[...],
                preferred_element_type=jnp.float32) + b2_ref[...]
    o_ref[...] = jax.nn.sigmoid(y)


def _recurrent_kernel(xg1_hbm, g1wbd_ref, g1bhh_ref,
                      g2w_ref, g2b_ref, g2wbd_ref, g2bhh_ref, o_ref,
                      xg1_ref, h_ref, sem, *, T, H, unroll, chunks=8):
    Bt = xg1_ref.shape[1]
    Tc = T // chunks

    b = pl.program_id(0)
    cp = pltpu.make_async_copy(
        xg1_hbm.at[:, pl.ds(b * Bt, Bt), :], xg1_ref, sem)
    cp.start()
    cp.wait()

    _gru_steps(xg1_ref, g1wbd_ref[...], g1bhh_ref[...], h_ref,
               T=T, H=H, Bt=Bt, unroll=unroll)

    for c in range(chunks):
        hc = h_ref[c * Tc:(c + 1) * Tc].reshape(Tc * Bt, 2 * H)
        xg2 = jnp.dot(hc, g2w_ref[...],
                      preferred_element_type=jnp.float32) + g2b_ref[...]
        xg1_ref[c * Tc:(c + 1) * Tc] = (
            xg2.reshape(Tc, Bt, 6 * H).astype(jnp.bfloat16))
    _gru_steps(xg1_ref, g2wbd_ref[...], g2bhh_ref[...], h_ref,
               T=T, H=H, Bt=Bt, unroll=unroll)

    for c in range(chunks):
        o_ref[c * Tc:(c + 1) * Tc] = (
            h_ref[c * Tc:(c + 1) * Tc].astype(jnp.bfloat16))


def _recurrent_stack(xg1, g1_wbd, g1_bhh, g2_wiht, g2_bih, g2_wbd, g2_bhh,
                     fc1w, fc1b, fc2w, fc2b, *, bt=128):
    T, B, _ = xg1.shape
    H = g1_wbd.shape[0] // 2
    n_fc = fc1w.shape[1]
    n_out = fc2w.shape[1]

    full = lambda shape: pl.BlockSpec(shape, lambda b: tuple(0 for _ in shape))
    h2 = pl.pallas_call(
        functools.partial(_recurrent_kernel, T=T, H=H, unroll=8),
        out_shape=jax.ShapeDtypeStruct((T, B, 2 * H), jnp.bfloat16),
        grid=(B // bt,),
        in_specs=[pl.BlockSpec(memory_space=pl.ANY),
                  full((2 * H, 6 * H)), full((1, 6 * H)),
                  full((2 * H, 6 * H)), full((1, 6 * H)),
                  full((2 * H, 6 * H)), full((1, 6 * H))],
        out_specs=pl.BlockSpec((T, bt, 2 * H), lambda b: (0, b, 0)),
        scratch_shapes=[pltpu.VMEM((T, bt, 6 * H), jnp.bfloat16),
                        pltpu.VMEM((T, bt, 2 * H), jnp.float32),
                        pltpu.SemaphoreType.DMA],
        compiler_params=pltpu.CompilerParams(
            dimension_semantics=("parallel",),
            vmem_limit_bytes=62 * 1024 * 1024),
    )(xg1, g1_wbd.astype(jnp.float32), g1_bhh,
      g2_wiht.astype(jnp.float32), g2_bih,
      g2_wbd.astype(jnp.float32), g2_bhh)

    m = T * B
    tile_m = 4096
    out2 = pl.pallas_call(
        functools.partial(_head_kernel, n_fc=n_fc),
        out_shape=jax.ShapeDtypeStruct((m, n_out), jnp.float32),
        grid=(m // tile_m,),
        in_specs=[pl.BlockSpec((tile_m, 2 * H), lambda i: (i, 0)),
                  full((2 * H, n_fc)), full((1, n_fc)),
                  full((n_fc, n_out)), full((1, n_out))],
        out_specs=pl.BlockSpec((tile_m, n_out), lambda i: (i, 0)),
        compiler_params=pltpu.CompilerParams(
            dimension_semantics=("parallel",)),
    )(h2.reshape(m, 2 * H), fc1w, fc1b, fc2w, fc2b)
    return out2.reshape(T, B, n_out)


def kernel(x, wk1, s1, t1, wk2, s2, t2, wk3, s3, t3,
           g1_wiht, g1_bih, g1_wbd, g1_bhh,
           g2_wiht, g2_bih, g2_wbd, g2_bhh,
           fc1w, fc1b, fc2w, fc2b):
    H = g1_wbd.shape[0] // 2
    p = np.concatenate([np.arange(0, H), np.arange(3 * H, 4 * H),
                        np.arange(H, 2 * H), np.arange(4 * H, 5 * H),
                        np.arange(2 * H, 3 * H), np.arange(5 * H, 6 * H)])
    xg1 = _conv_stack(x, wk1, s1, t1, wk2, s2, t2, wk3, s3, t3,
                      g1_wiht[:, p], g1_bih[:, p])
    out_tm = _recurrent_stack(xg1, g1_wbd[:, p], g1_bhh[:, p],
                              g2_wiht[:, p], g2_bih[:, p],
                              g2_wbd[:, p], g2_bhh[:, p],
                              fc1w, fc1b, fc2w, fc2b)
    return jnp.transpose(out_tm, (1, 0, 2))


# speedup vs baseline: 1.0168x; 1.0004x over previous
import functools

import numpy as np
import jax
import jax.numpy as jnp
from jax import lax
from jax.experimental import pallas as pl
from jax.experimental.pallas import tpu as pltpu


def _banded_weight(w_taps, win, wout):
    cin, cout = w_taps.shape[1], w_taps.shape[2]
    per_kh = []
    for kh in range(3):
        acc = jnp.zeros((win, cin, wout, cout), jnp.float32)
        for kw in range(3):
            sel = jnp.eye(win, wout, k=-kw, dtype=jnp.float32)
            tap = w_taps[kh * 3 + kw].astype(jnp.float32)
            acc = acc + jnp.einsum("io,cd->icod", sel, tap)
        per_kh.append(acc.reshape(win * cin, wout * cout))
    return jnp.concatenate(per_kh, axis=0).astype(jnp.bfloat16)


def _banded_weight_c1(wk1, win, wout):
    return _banded_weight(wk1[:, None, :], win, wout)


def _pool_max_lanes(y, groups, pool, c):
    outs = []
    for g in range(groups):
        m = y[:, (g * pool) * c:(g * pool) * c + c]
        for p in range(1, pool):
            m = jnp.maximum(m, y[:, (g * pool + p) * c:(g * pool + p) * c + c])
        outs.append(m)
    return jnp.concatenate(outs, axis=1)


def _conv_banded_grouped(cp, w, t, wout, *, T):
    outs = []
    for g in range(wout // 2):
        y = sum(jnp.dot(cp[kh:kh + T, g * 256:g * 256 + 512],
                        w[kh * 512:(kh + 1) * 512],
                        preferred_element_type=jnp.float32)
                for kh in range(3))
        outs.append(y)
    y = jnp.concatenate(outs, axis=1)
    y = _pool_max_lanes(y, wout // 2, 2, 128)
    return jnp.maximum(y + t, 0.0).astype(jnp.bfloat16)


def _conv_stack_kernel(xp_ref, w1_ref, t1_ref, w2_ref, t2_ref,
                       w3_ref, t3_ref, gw_ref, gb_ref, o_ref, *, T, bt):
    w1 = w1_ref[...]
    w2 = w2_ref[...]
    w3 = w3_ref[...]
    gw = gw_ref[...]

    for i in range(bt):
        xi = xp_ref[i]
        a = jnp.concatenate([xi[0:T], xi[1:T + 1], xi[2:T + 2]],
                            axis=1).astype(jnp.bfloat16)
        y = jnp.dot(a, w1, preferred_element_type=jnp.float32)
        y = _pool_max_lanes(y, 8, 5, 128)
        c1 = jnp.maximum(y + t1_ref[...], 0.0).astype(jnp.bfloat16)
        c1 = jnp.pad(c1, ((1, 1), (128, 128)))

        c2 = _conv_banded_grouped(c1, w2, t2_ref[...], 8, T=T)
        c2 = jnp.pad(c2, ((1, 1), (128, 128)))

        c3 = _conv_banded_grouped(c2, w3, t3_ref[...], 4, T=T)
        xg = jnp.dot(c3, gw, preferred_element_type=jnp.float32) + gb_ref[...]
        o_ref[:, i, :] = xg.astype(o_ref.dtype)


def _conv_stack(x, wk1, s1, t1, wk2, s2, t2, wk3, s3, t3, g1w, g1b, *, bt=8):
    B, T, F = x.shape
    xp = jnp.pad(x, ((0, 0), (1, 1), (1, 1)))
    GH = g1w.shape[1]

    w1 = _banded_weight_c1(wk1.astype(jnp.float32) * s1, F + 2, F)
    w2 = _banded_weight(wk2.astype(jnp.float32) * s2, 4, 2)
    w3 = _banded_weight(wk3.astype(jnp.float32) * s3, 4, 2)
    t1t = jnp.tile(t1, (1, F // 5))
    t2t = jnp.tile(t2, (1, 4))
    t3t = jnp.tile(t3, (1, 2))

    kern = functools.partial(_conv_stack_kernel, T=T, bt=bt)
    full = lambda shape: pl.BlockSpec(shape, lambda b: tuple(0 for _ in shape))
    return pl.pallas_call(
        kern,
        out_shape=jax.ShapeDtypeStruct((T, B, GH), jnp.bfloat16),
        grid=(B // bt,),
        in_specs=[pl.BlockSpec((bt, T + 2, F + 2), lambda b: (b, 0, 0)),
                  full((126, 5120)), full((1, 1024)),
                  full((1536, 256)), full((1, 512)),
                  full((1536, 256)), full((1, 256)),
                  full((256, GH)), full((1, GH))],
        out_specs=pl.BlockSpec((T, bt, GH), lambda b: (0, b, 0)),
        compiler_params=pltpu.CompilerParams(
            dimension_semantics=("parallel",),
            vmem_limit_bytes=60 * 1024 * 1024),
    )(xp, w1, t1t, w2, t2t, w3, t3t, g1w.astype(jnp.bfloat16), g1b)


def _gru_steps(xg_ref, w_bd, b_hh, h_out_ref, *, T, H, Bt, unroll):
    H2, H4 = 2 * H, 4 * H
    lane = lax.broadcasted_iota(jnp.int32, (1, 6 * H), 1)
    fwd_lane = (lane % H2) < H

    def step(t, h):
        tb = T - 1 - t
        xgm = jnp.where(fwd_lane, xg_ref[t], xg_ref[tb])
        hg = jnp.dot(h, w_bd, preferred_element_type=jnp.float32) + b_hh
        rz = jax.nn.sigmoid(xgm[:, 0:H4] + hg[:, 0:H4])
        n = jnp.tanh(xgm[:, H4:] + rz[:, 0:H2] * hg[:, H4:])
        z = rz[:, H2:H4]
        h_new = (1.0 - z) * n + z * h
        h_out_ref[pl.ds(t, 1), :, 0:H] = h_new[None, :, 0:H]
        h_out_ref[pl.ds(tb, 1), :, H:H2] = h_new[None, :, H:H2]
        return h_new

    lax.fori_loop(0, T, step, jnp.zeros((Bt, H2), jnp.float32), unroll=unroll)


def _head_kernel(h_ref, w1_ref, b1_ref, w2_ref, b2_ref, o_ref, *, n_fc):
    h1 = jnp.dot(h_ref[...], w1_ref[...],
                 preferred_element_type=jnp.float32) + b1_ref[...]
    h1 = jnp.maximum(h1, 0.0)
    y = jnp.dot(h1.astype(jnp.bfloat16), w2_ref[...],
                preferred_element_type=jnp.float32) + b2_ref[...]
    o_ref[...] = jax.nn.sigmoid(y)


def _recurrent_kernel(xg1_hbm, g1wbd_ref, g1bhh_ref,
                      g2w_ref, g2b_ref, g2wbd_ref, g2bhh_ref, o_ref,
                      xg1_ref, h_ref, sem, *, T, H, unroll, chunks=8):
    Bt = xg1_ref.shape[1]
    Tc = T // chunks

    b = pl.program_id(0)
    cp = pltpu.make_async_copy(
        xg1_hbm.at[:, pl.ds(b * Bt, Bt), :], xg1_ref, sem)
    cp.start()
    cp.wait()

    _gru_steps(xg1_ref, g1wbd_ref[...], g1bhh_ref[...], h_ref,
               T=T, H=H, Bt=Bt, unroll=unroll)

    for c in range(chunks):
        hc = h_ref[c * Tc:(c + 1) * Tc].reshape(Tc * Bt, 2 * H)
        xg2 = jnp.dot(hc, g2w_ref[...],
                      preferred_element_type=jnp.float32) + g2b_ref[...]
        xg1_ref[c * Tc:(c + 1) * Tc] = (
            xg2.reshape(Tc, Bt, 6 * H).astype(jnp.bfloat16))
    _gru_steps(xg1_ref, g2wbd_ref[...], g2bhh_ref[...], h_ref,
               T=T, H=H, Bt=Bt, unroll=unroll)

    for c in range(chunks):
        o_ref[c * Tc:(c + 1) * Tc] = (
            h_ref[c * Tc:(c + 1) * Tc].astype(jnp.bfloat16))


def _recurrent_stack(xg1, g1_wbd, g1_bhh, g2_wiht, g2_bih, g2_wbd, g2_bhh,
                     fc1w, fc1b, fc2w, fc2b, *, bt=128):
    T, B, _ = xg1.shape
    H = g1_wbd.shape[0] // 2
    n_fc = fc1w.shape[1]
    n_out = fc2w.shape[1]

    full = lambda shape: pl.BlockSpec(shape, lambda b: tuple(0 for _ in shape))
    h2 = pl.pallas_call(
        functools.partial(_recurrent_kernel, T=T, H=H, unroll=8),
        out_shape=jax.ShapeDtypeStruct((T, B, 2 * H), jnp.bfloat16),
        grid=(B // bt,),
        in_specs=[pl.BlockSpec(memory_space=pl.ANY),
                  full((2 * H, 6 * H)), full((1, 6 * H)),
                  full((2 * H, 6 * H)), full((1, 6 * H)),
                  full((2 * H, 6 * H)), full((1, 6 * H))],
        out_specs=pl.BlockSpec((T, bt, 2 * H), lambda b: (0, b, 0)),
        scratch_shapes=[pltpu.VMEM((T, bt, 6 * H), jnp.bfloat16),
                        pltpu.VMEM((T, bt, 2 * H), jnp.float32),
                        pltpu.SemaphoreType.DMA],
        compiler_params=pltpu.CompilerParams(
            dimension_semantics=("parallel",),
            vmem_limit_bytes=62 * 1024 * 1024),
    )(xg1, g1_wbd.astype(jnp.float32), g1_bhh,
      g2_wiht.astype(jnp.float32), g2_bih,
      g2_wbd.astype(jnp.float32), g2_bhh)

    m = T * B
    tile_m = 4096
    out2 = pl.pallas_call(
        functools.partial(_head_kernel, n_fc=n_fc),
        out_shape=jax.ShapeDtypeStruct((m, n_out), jnp.float32),
        grid=(m // tile_m,),
        in_specs=[pl.BlockSpec((tile_m, 2 * H), lambda i: (i, 0)),
                  full((2 * H, n_fc)), full((1, n_fc)),
                  full((n_fc, n_out)), full((1, n_out))],
        out_specs=pl.BlockSpec((tile_m, n_out), lambda i: (i, 0)),
        compiler_params=pltpu.CompilerParams(
            dimension_semantics=("parallel",)),
    )(h2.reshape(m, 2 * H), fc1w, fc1b, fc2w, fc2b)
    return out2.reshape(T, B, n_out)


def kernel(x, wk1, s1, t1, wk2, s2, t2, wk3, s3, t3,
           g1_wiht, g1_bih, g1_wbd, g1_bhh,
           g2_wiht, g2_bih, g2_wbd, g2_bhh,
           fc1w, fc1b, fc2w, fc2b):
    H = g1_wbd.shape[0] // 2
    p = np.concatenate([np.arange(0, H), np.arange(3 * H, 4 * H),
                        np.arange(H, 2 * H), np.arange(4 * H, 5 * H),
                        np.arange(2 * H, 3 * H), np.arange(5 * H, 6 * H)])
    xg1 = _conv_stack(x, wk1, s1, t1, wk2, s2, t2, wk3, s3, t3,
                      g1_wiht[:, p], g1_bih[:, p])
    out_tm = _recurrent_stack(xg1, g1_wbd[:, p], g1_bhh[:, p],
                              g2_wiht[:, p], g2_bih[:, p],
                              g2_wbd[:, p], g2_bhh[:, p],
                              fc1w, fc1b, fc2w, fc2b)
    return jnp.transpose(out_tm, (1, 0, 2))
